# Initial kernel scaffold; baseline (speedup 1.0000x reference)
import functools

import jax
import jax.numpy as jnp
from jax import lax
from jax.experimental import pallas as pl
from jax.experimental.pallas import tpu as pltpu
from jax.experimental.pallas import tpu_sc as plsc

N = 10000
E = 320000
F = 128
N_CLASSES = 64

NC = 2
NS = 16
NW = NC * NS
CHUNK = 128
N_PAD = 10016
ROWS_PER_SUB = N_PAD // NS
EPW = 10112
E_PAD = EPW * NW
N_CHUNKS = EPW // CHUNK
PAD_ROWS = 16


def _sc_degree(dst_pad, ones_blk, zeros_blk):
    mesh = plsc.VectorSubcoreMesh(core_axis_name="c", subcore_axis_name="s")

    @functools.partial(
        pl.kernel,
        mesh=mesh,
        out_type=jax.ShapeDtypeStruct((NC, N_PAD, 16), jnp.float32),
        scratch_types=[
            pltpu.VMEM((CHUNK,), jnp.int32),
            pltpu.VMEM((CHUNK, 16), jnp.float32),
            pltpu.VMEM((ROWS_PER_SUB, 16), jnp.float32),
            pltpu.VMEM_SHARED((N_PAD, 16), jnp.float32),
        ],
    )
    def k(dst_hbm, ones_hbm, zeros_hbm, out_hbm, didx, ones_v, zrows_v, acc):
        cid = lax.axis_index("c")
        sid = lax.axis_index("s")
        pltpu.sync_copy(ones_hbm, ones_v)
        pltpu.sync_copy(zeros_hbm, zrows_v)
        pltpu.sync_copy(zrows_v, acc.at[pl.ds(sid * ROWS_PER_SUB, ROWS_PER_SUB)])
        plsc.subcore_barrier()

        wid = cid * NS + sid
        ebase = wid * EPW

        @pl.loop(0, N_CHUNKS)
        def _(i):
            pltpu.sync_copy(dst_hbm.at[pl.ds(ebase + i * CHUNK, CHUNK)], didx)
            pltpu.sync_copy(ones_v, acc.at[didx], add=True)

        plsc.subcore_barrier()
        rows = pl.ds(sid * ROWS_PER_SUB, ROWS_PER_SUB)
        pltpu.sync_copy(acc.at[rows], out_hbm.at[cid].at[rows])

    return k(dst_pad, ones_blk, zeros_blk)


def _sc_propagate(g, src_pad, dst_pad, zeros_blk):
    mesh = plsc.VectorSubcoreMesh(core_axis_name="c", subcore_axis_name="s")

    @functools.partial(
        pl.kernel,
        mesh=mesh,
        out_type=jax.ShapeDtypeStruct((NC, N_PAD, F), jnp.float32),
        scratch_types=[
            pltpu.VMEM((CHUNK,), jnp.int32),
            pltpu.VMEM((CHUNK,), jnp.int32),
            pltpu.VMEM((CHUNK,), jnp.int32),
            pltpu.VMEM((CHUNK,), jnp.int32),
            pltpu.VMEM((CHUNK, F), jnp.float32),
            pltpu.VMEM((CHUNK, F), jnp.float32),
            pltpu.VMEM((ROWS_PER_SUB, F), jnp.float32),
            pltpu.VMEM_SHARED((N_PAD, F), jnp.float32),
            pltpu.SemaphoreType.DMA,
            pltpu.SemaphoreType.DMA,
        ],
    )
    def k(g_hbm, src_hbm, dst_hbm, zeros_hbm, out_hbm,
          sidx0, sidx1, didx0, didx1, rows0, rows1, zrows_v, acc, sem0, sem1):
        cid = lax.axis_index("c")
        sid = lax.axis_index("s")
        pltpu.sync_copy(zeros_hbm, zrows_v)
        pltpu.sync_copy(zrows_v, acc.at[pl.ds(sid * ROWS_PER_SUB, ROWS_PER_SUB)])
        plsc.subcore_barrier()

        wid = cid * NS + sid
        ebase = wid * EPW

        pltpu.sync_copy(src_hbm.at[pl.ds(ebase, CHUNK)], sidx0)
        pltpu.sync_copy(dst_hbm.at[pl.ds(ebase, CHUNK)], didx0)
        pltpu.async_copy(g_hbm.at[sidx0], rows0, sem0)
        pltpu.sync_copy(src_hbm.at[pl.ds(ebase + CHUNK, CHUNK)], sidx1)
        pltpu.sync_copy(dst_hbm.at[pl.ds(ebase + CHUNK, CHUNK)], didx1)
        pltpu.async_copy(g_hbm.at[sidx1], rows1, sem1)

        @pl.loop(0, N_CHUNKS, step=2)
        def _(i):
            pltpu.make_async_copy(g_hbm.at[sidx0], rows0, sem0).wait()
            pltpu.sync_copy(rows0, acc.at[didx0], add=True)

            @pl.when(i + 2 < N_CHUNKS)
            def _():
                b = ebase + (i + 2) * CHUNK
                pltpu.sync_copy(src_hbm.at[pl.ds(b, CHUNK)], sidx0)
                pltpu.sync_copy(dst_hbm.at[pl.ds(b, CHUNK)], didx0)
                pltpu.async_copy(g_hbm.at[sidx0], rows0, sem0)

            pltpu.make_async_copy(g_hbm.at[sidx1], rows1, sem1).wait()
            pltpu.sync_copy(rows1, acc.at[didx1], add=True)

            @pl.when(i + 3 < N_CHUNKS)
            def _():
                b = ebase + (i + 3) * CHUNK
                pltpu.sync_copy(src_hbm.at[pl.ds(b, CHUNK)], sidx1)
                pltpu.sync_copy(dst_hbm.at[pl.ds(b, CHUNK)], didx1)
                pltpu.async_copy(g_hbm.at[sidx1], rows1, sem1)

        plsc.subcore_barrier()
        rows = pl.ds(sid * ROWS_PER_SUB, ROWS_PER_SUB)
        pltpu.sync_copy(acc.at[rows], out_hbm.at[cid].at[rows])

    return k(g, src_pad, dst_pad, zeros_blk)


def _tc_prep(deg2, features):
    def body(deg_ref, x_ref, norm_ref, nsq_ref, g0_ref):
        deg = deg_ref[0, :, 0:1] + deg_ref[1, :, 0:1]
        row = lax.broadcasted_iota(jnp.int32, (N_PAD, 1), 0)
        norm = jnp.where(row < N, lax.rsqrt(jnp.maximum(deg, 1.0)), 0.0)
        norm_ref[...] = norm
        nsq_ref[...] = norm * norm
        g0_ref[0:N, :] = x_ref[...] * norm[0:N]
        g0_ref[N:N_PAD, :] = jnp.zeros((N_PAD - N, F), jnp.float32)

    return pl.pallas_call(
        body,
        out_shape=(
            jax.ShapeDtypeStruct((N_PAD, 1), jnp.float32),
            jax.ShapeDtypeStruct((N_PAD, 1), jnp.float32),
            jax.ShapeDtypeStruct((N_PAD, F), jnp.float32),
        ),
    )(deg2, features)


def _tc_mid(a1p, normsq):
    def body(a_ref, nsq_ref, a1_ref, g1_ref):
        a1 = a_ref[0] + a_ref[1]
        a1_ref[...] = a1
        g1_ref[...] = a1 * nsq_ref[...]

    return pl.pallas_call(
        body,
        out_shape=(
            jax.ShapeDtypeStruct((N_PAD, F), jnp.float32),
            jax.ShapeDtypeStruct((N_PAD, F), jnp.float32),
        ),
    )(a1p, normsq)


def _tc_layer(x, a1, a2p, norm, Wa, Wb, Wc, b, relu, want_g):
    fout = Wa.shape[1]

    def body(x_ref, a1_ref, a2_ref, n_ref, wa_ref, wb_ref, wc_ref, b_ref, *outs):
        nrm = n_ref[...]
        z1 = a1_ref[...] * nrm
        z2 = (a2_ref[0] + a2_ref[1]) * nrm
        acc = jnp.dot(x_ref[...], wa_ref[...], preferred_element_type=jnp.float32,
                      precision=lax.Precision.HIGHEST)
        acc += jnp.dot(z1, wb_ref[...], preferred_element_type=jnp.float32,
                       precision=lax.Precision.HIGHEST)
        acc += jnp.dot(z2, wc_ref[...], preferred_element_type=jnp.float32,
                       precision=lax.Precision.HIGHEST)
        acc += b_ref[...]
        if relu:
            acc = jnp.maximum(acc, 0.0)
        outs[0][...] = acc
        if want_g:
            outs[1][...] = acc * nrm

    out_shape = [jax.ShapeDtypeStruct((N_PAD, fout), jnp.float32)]
    if want_g:
        out_shape.append(jax.ShapeDtypeStruct((N_PAD, fout), jnp.float32))

    return pl.pallas_call(body, out_shape=tuple(out_shape))(
        x, a1, a2p, norm, Wa, Wb, Wc, b)


def kernel(features, edge_index, W0, b0, W1, b1, W2, b2):
    src = edge_index[0].astype(jnp.int32)
    dst = edge_index[1].astype(jnp.int32)
    npad_e = E_PAD - E
    pad_idx = (N + (jnp.arange(npad_e, dtype=jnp.int32) % PAD_ROWS))
    src_pad = jnp.concatenate([src, pad_idx])
    dst_pad = jnp.concatenate([dst, pad_idx])

    ones_blk = jnp.ones((CHUNK, 16), jnp.float32)
    zeros16 = jnp.zeros((ROWS_PER_SUB, 16), jnp.float32)
    zeros128 = jnp.zeros((ROWS_PER_SUB, F), jnp.float32)

    deg2 = _sc_degree(dst_pad, ones_blk, zeros16)
    norm, normsq, g = _tc_prep(deg2, features)

    x = jnp.pad(features, ((0, N_PAD - N), (0, 0)))
    params = [(W0, b0, True), (W1, b1, True), (W2, b2, False)]
    for li, (W, b, relu) in enumerate(params):
        fin = x.shape[1]
        Wa, Wb, Wc = W[0:fin], W[fin:2 * fin], W[2 * fin:3 * fin]
        a1p = _sc_propagate(g, src_pad, dst_pad, zeros128)
        a1, g1 = _tc_mid(a1p, normsq)
        a2p = _sc_propagate(g1, src_pad, dst_pad, zeros128)
        want_g = li < 2
        outs = _tc_layer(x, a1, a2p, norm, Wa, Wb, Wc, b.reshape(1, -1),
                         relu, want_g)
        if want_g:
            x, g = outs
        else:
            x = outs[0]
    return x[:N]


# speedup vs baseline: 4.9727x; 4.9727x over previous
import functools

import jax
import jax.numpy as jnp
from jax import lax
from jax.experimental import pallas as pl
from jax.experimental.pallas import tpu as pltpu
from jax.experimental.pallas import tpu_sc as plsc

N = 10000
E = 320000
F = 128
N_CLASSES = 64

NC = 2
NS = 16
NW = NC * NS
CHUNK = 128
N_PAD = 10112
ROWS_PER_SUB = N_PAD // NS
EPW = 10240
E_PAD = EPW * NW
N_CHUNKS = EPW // CHUNK
PAD_ROWS = 112
RB = 1264


def _sc_degree(dst_pad, ones_blk, zeros_blk):
    mesh = plsc.VectorSubcoreMesh(core_axis_name="c", subcore_axis_name="s")

    @functools.partial(
        pl.kernel,
        mesh=mesh,
        out_type=jax.ShapeDtypeStruct((NC, N_PAD, F), jnp.float32),
        scratch_types=[
            pltpu.VMEM((CHUNK,), jnp.int32),
            pltpu.VMEM((CHUNK, F), jnp.float32),
            pltpu.VMEM_SHARED((N_PAD, F), jnp.float32),
        ],
    )
    def k(dst_hbm, ones_hbm, zeros_hbm, out_hbm, didx, ones_v, acc):
        cid = lax.axis_index("c")
        sid = lax.axis_index("s")
        pltpu.sync_copy(ones_hbm, ones_v)
        pltpu.sync_copy(zeros_hbm, acc.at[pl.ds(sid * ROWS_PER_SUB, ROWS_PER_SUB)])
        plsc.subcore_barrier()

        wid = cid * NS + sid
        ebase = wid * EPW

        @pl.loop(0, N_CHUNKS)
        def _(i):
            pltpu.sync_copy(dst_hbm.at[pl.ds(ebase + i * CHUNK, CHUNK)], didx)
            pltpu.sync_copy(ones_v, acc.at[didx], add=True)

        plsc.subcore_barrier()
        rows = pl.ds(sid * ROWS_PER_SUB, ROWS_PER_SUB)
        pltpu.sync_copy(acc.at[rows], out_hbm.at[cid].at[rows])

    return k(dst_pad, ones_blk, zeros_blk)


def _sc_propagate(g, src_pad, dst_pad, zeros_blk):
    mesh = plsc.VectorSubcoreMesh(core_axis_name="c", subcore_axis_name="s")

    @functools.partial(
        pl.kernel,
        mesh=mesh,
        out_type=jax.ShapeDtypeStruct((NC, N_PAD, F), jnp.float32),
        scratch_types=[
            pltpu.VMEM((CHUNK,), jnp.int32),
            pltpu.VMEM((CHUNK,), jnp.int32),
            pltpu.VMEM((CHUNK,), jnp.int32),
            pltpu.VMEM((CHUNK,), jnp.int32),
            pltpu.VMEM((CHUNK, F), jnp.float32),
            pltpu.VMEM((CHUNK, F), jnp.float32),
            pltpu.VMEM_SHARED((N_PAD, F), jnp.float32),
            pltpu.SemaphoreType.DMA,
            pltpu.SemaphoreType.DMA,
        ],
    )
    def k(g_hbm, src_hbm, dst_hbm, zeros_hbm, out_hbm,
          sidx0, sidx1, didx0, didx1, rows0, rows1, acc, sem0, sem1):
        cid = lax.axis_index("c")
        sid = lax.axis_index("s")
        pltpu.sync_copy(zeros_hbm, acc.at[pl.ds(sid * ROWS_PER_SUB, ROWS_PER_SUB)])
        plsc.subcore_barrier()

        wid = cid * NS + sid
        ebase = wid * EPW

        @pl.loop(0, N_CHUNKS, step=2)
        def _(i):
            b0 = ebase + i * CHUNK
            pltpu.sync_copy(src_hbm.at[pl.ds(b0, CHUNK)], sidx0)
            pltpu.sync_copy(dst_hbm.at[pl.ds(b0, CHUNK)], didx0)
            pltpu.async_copy(g_hbm.at[sidx0], rows0, sem0).wait()
            pltpu.sync_copy(rows0, acc.at[didx0], add=True)

            b1 = ebase + (i + 1) * CHUNK
            pltpu.sync_copy(src_hbm.at[pl.ds(b1, CHUNK)], sidx1)
            pltpu.sync_copy(dst_hbm.at[pl.ds(b1, CHUNK)], didx1)
            pltpu.async_copy(g_hbm.at[sidx1], rows1, sem1).wait()
            pltpu.sync_copy(rows1, acc.at[didx1], add=True)

        plsc.subcore_barrier()
        rows = pl.ds(sid * ROWS_PER_SUB, ROWS_PER_SUB)
        pltpu.sync_copy(acc.at[rows], out_hbm.at[cid].at[rows])

    return k(g, src_pad, dst_pad, zeros_blk)


def _tc_prep(deg2, features):
    def body(deg_ref, x_ref, norm_ref, nsq_ref, g0_ref):
        deg = deg_ref[0, :, 0:1] + deg_ref[1, :, 0:1]
        row = lax.broadcasted_iota(jnp.int32, (N_PAD, 1), 0)
        norm = jnp.where(row < N, lax.rsqrt(jnp.maximum(deg, 1.0)), 0.0)
        norm_ref[...] = norm
        nsq_ref[...] = norm * norm
        g0_ref[0:N, :] = x_ref[...] * norm[0:N]
        g0_ref[N:N_PAD, :] = jnp.zeros((N_PAD - N, F), jnp.float32)

    return pl.pallas_call(
        body,
        out_shape=(
            jax.ShapeDtypeStruct((N_PAD, 1), jnp.float32),
            jax.ShapeDtypeStruct((N_PAD, 1), jnp.float32),
            jax.ShapeDtypeStruct((N_PAD, F), jnp.float32),
        ),
    )(deg2, features)


def _tc_mid(a1p, normsq):
    def body(a_ref, nsq_ref, a1_ref, g1_ref):
        a1 = a_ref[0] + a_ref[1]
        a1_ref[...] = a1
        g1_ref[...] = a1 * nsq_ref[...]

    return pl.pallas_call(
        body,
        grid=(N_PAD // RB,),
        in_specs=[
            pl.BlockSpec((NC, RB, F), lambda i: (0, i, 0)),
            pl.BlockSpec((RB, 1), lambda i: (i, 0)),
        ],
        out_specs=[
            pl.BlockSpec((RB, F), lambda i: (i, 0)),
            pl.BlockSpec((RB, F), lambda i: (i, 0)),
        ],
        out_shape=(
            jax.ShapeDtypeStruct((N_PAD, F), jnp.float32),
            jax.ShapeDtypeStruct((N_PAD, F), jnp.float32),
        ),
    )(a1p, normsq)


def _tc_layer(x, a1, a2p, norm, Wa, Wb, Wc, b, relu, want_g):
    fout = Wa.shape[1]

    def body(x_ref, a1_ref, a2_ref, n_ref, wa_ref, wb_ref, wc_ref, b_ref, *outs):
        nrm = n_ref[...]
        z1 = a1_ref[...] * nrm
        z2 = (a2_ref[0] + a2_ref[1]) * nrm
        acc = jnp.dot(x_ref[...], wa_ref[...], preferred_element_type=jnp.float32,
                      precision=lax.Precision.HIGHEST)
        acc += jnp.dot(z1, wb_ref[...], preferred_element_type=jnp.float32,
                       precision=lax.Precision.HIGHEST)
        acc += jnp.dot(z2, wc_ref[...], preferred_element_type=jnp.float32,
                       precision=lax.Precision.HIGHEST)
        acc += b_ref[...]
        if relu:
            acc = jnp.maximum(acc, 0.0)
        outs[0][...] = acc
        if want_g:
            outs[1][...] = acc * nrm

    out_shape = [jax.ShapeDtypeStruct((N_PAD, fout), jnp.float32)]
    if want_g:
        out_shape.append(jax.ShapeDtypeStruct((N_PAD, fout), jnp.float32))

    fin = x.shape[1]
    return pl.pallas_call(
        body,
        grid=(N_PAD // RB,),
        in_specs=[
            pl.BlockSpec((RB, fin), lambda i: (i, 0)),
            pl.BlockSpec((RB, F), lambda i: (i, 0)),
            pl.BlockSpec((NC, RB, F), lambda i: (0, i, 0)),
            pl.BlockSpec((RB, 1), lambda i: (i, 0)),
            pl.BlockSpec((fin, fout), lambda i: (0, 0)),
            pl.BlockSpec((F, fout), lambda i: (0, 0)),
            pl.BlockSpec((F, fout), lambda i: (0, 0)),
            pl.BlockSpec((1, fout), lambda i: (0, 0)),
        ],
        out_specs=[pl.BlockSpec((RB, fout), lambda i: (i, 0))
                   for _ in out_shape],
        out_shape=tuple(out_shape),
    )(x, a1, a2p, norm, Wa, Wb, Wc, b)


def kernel(features, edge_index, W0, b0, W1, b1, W2, b2):
    src = edge_index[0].astype(jnp.int32)
    dst = edge_index[1].astype(jnp.int32)
    npad_e = E_PAD - E
    pad_idx = (N + (jnp.arange(npad_e, dtype=jnp.int32) % PAD_ROWS))
    src_pad = jnp.concatenate([src, pad_idx])
    dst_pad = jnp.concatenate([dst, pad_idx])

    ones_blk = jnp.ones((CHUNK, F), jnp.float32)
    zeros128 = jnp.zeros((ROWS_PER_SUB, F), jnp.float32)

    deg2 = _sc_degree(dst_pad, ones_blk, zeros128)
    norm, normsq, g = _tc_prep(deg2, features)

    x = jnp.pad(features, ((0, N_PAD - N), (0, 0)))
    params = [(W0, b0, True), (W1, b1, True), (W2, b2, False)]
    for li, (W, b, relu) in enumerate(params):
        fin = x.shape[1]
        Wa, Wb, Wc = W[0:fin], W[fin:2 * fin], W[2 * fin:3 * fin]
        a1p = _sc_propagate(g, src_pad, dst_pad, zeros128)
        a1, g1 = _tc_mid(a1p, normsq)
        a2p = _sc_propagate(g1, src_pad, dst_pad, zeros128)
        want_g = li < 2
        outs = _tc_layer(x, a1, a2p, norm, Wa, Wb, Wc, b.reshape(1, -1),
                         relu, want_g)
        if want_g:
            x, g = outs
        else:
            x = outs[0]
    return x[:N]


# speedup vs baseline: 7.6045x; 1.5293x over previous
import functools

import jax
import jax.numpy as jnp
from jax import lax
from jax.experimental import pallas as pl
from jax.experimental.pallas import tpu as pltpu
from jax.experimental.pallas import tpu_sc as plsc

N = 10000
E = 320000
F = 128
N_CLASSES = 64

NC = 2
NS = 16
NW = NC * NS
CHUNK = 128
N_PAD = 10112
ROWS_PER_SUB = N_PAD // NS
EPW = 10240
E_PAD = EPW * NW
N_CHUNKS = EPW // CHUNK
PAD_ROWS = 112
RB = 1264


HALF = 40


def _sc_degree(dst_pad, ones_blk, zeros_blk):
    mesh = plsc.VectorSubcoreMesh(core_axis_name="c", subcore_axis_name="s")

    @functools.partial(
        pl.kernel,
        mesh=mesh,
        out_type=jax.ShapeDtypeStruct((NC, N_PAD, F), jnp.float32),
        scratch_types=[
            pltpu.VMEM((HALF, CHUNK), jnp.int32),
            pltpu.VMEM((CHUNK, F), jnp.float32),
            pltpu.VMEM_SHARED((N_PAD, F), jnp.float32),
        ],
    )
    def k(dst_hbm, ones_hbm, zeros_hbm, out_hbm, didx, ones_v, acc):
        cid = lax.axis_index("c")
        sid = lax.axis_index("s")
        pltpu.sync_copy(ones_hbm, ones_v)
        pltpu.sync_copy(zeros_hbm, acc.at[pl.ds(sid * ROWS_PER_SUB, ROWS_PER_SUB)])
        plsc.subcore_barrier()

        wid = cid * NS + sid

        for half in range(N_CHUNKS // HALF):
            pltpu.sync_copy(dst_hbm.at[wid].at[pl.ds(half * HALF, HALF)], didx)

            @pl.loop(0, HALF)
            def _(j):
                pltpu.sync_copy(ones_v, acc.at[didx.at[j]], add=True)

        plsc.subcore_barrier()
        rows = pl.ds(sid * ROWS_PER_SUB, ROWS_PER_SUB)
        pltpu.sync_copy(acc.at[rows], out_hbm.at[cid].at[rows])

    return k(dst_pad, ones_blk, zeros_blk)


def _sc_propagate(g, src_pad, dst_pad, zeros_blk):
    mesh = plsc.VectorSubcoreMesh(core_axis_name="c", subcore_axis_name="s")

    @functools.partial(
        pl.kernel,
        mesh=mesh,
        out_type=jax.ShapeDtypeStruct((NC, N_PAD, F), jnp.float32),
        scratch_types=[
            pltpu.VMEM((HALF * CHUNK,), jnp.int32),
            pltpu.VMEM((HALF, CHUNK), jnp.int32),
            pltpu.VMEM((CHUNK, F), jnp.float32),
            pltpu.VMEM((CHUNK, F), jnp.float32),
            pltpu.VMEM_SHARED((N_PAD, F), jnp.float32),
            pltpu.SemaphoreType.DMA,
            pltpu.SemaphoreType.DMA,
        ],
    )
    def k(g_hbm, src_hbm, dst_hbm, zeros_hbm, out_hbm,
          sidx, didx, rows0, rows1, acc, sem0, sem1):
        cid = lax.axis_index("c")
        sid = lax.axis_index("s")
        pltpu.sync_copy(zeros_hbm, acc.at[pl.ds(sid * ROWS_PER_SUB, ROWS_PER_SUB)])
        plsc.subcore_barrier()

        wid = cid * NS + sid
        ebase = wid * EPW

        for half in range(N_CHUNKS // HALF):
            pltpu.sync_copy(
                src_hbm.at[pl.ds(ebase + half * HALF * CHUNK, HALF * CHUNK)],
                sidx)
            pltpu.sync_copy(dst_hbm.at[wid].at[pl.ds(half * HALF, HALF)], didx)

            @pl.loop(0, HALF, step=2)
            def _(j):
                c0 = pltpu.async_copy(
                    g_hbm.at[sidx.at[pl.ds(j * CHUNK, CHUNK)]], rows0, sem0)
                c1 = pltpu.async_copy(
                    g_hbm.at[sidx.at[pl.ds((j + 1) * CHUNK, CHUNK)]], rows1, sem1)
                c0.wait()
                pltpu.sync_copy(rows0, acc.at[didx.at[j]], add=True)
                c1.wait()
                pltpu.sync_copy(rows1, acc.at[didx.at[j + 1]], add=True)

        plsc.subcore_barrier()
        rows = pl.ds(sid * ROWS_PER_SUB, ROWS_PER_SUB)
        pltpu.sync_copy(acc.at[rows], out_hbm.at[cid].at[rows])

    return k(g, src_pad, dst_pad, zeros_blk)


def _tc_prep(deg2, features):
    def body(deg_ref, x_ref, norm_ref, nsq_ref, g0_ref):
        deg = deg_ref[0, :, 0:1] + deg_ref[1, :, 0:1]
        row = lax.broadcasted_iota(jnp.int32, (N_PAD, 1), 0)
        norm = jnp.where(row < N, lax.rsqrt(jnp.maximum(deg, 1.0)), 0.0)
        norm_ref[...] = norm
        nsq_ref[...] = norm * norm
        g0_ref[0:N, :] = x_ref[...] * norm[0:N]
        g0_ref[N:N_PAD, :] = jnp.zeros((N_PAD - N, F), jnp.float32)

    return pl.pallas_call(
        body,
        out_shape=(
            jax.ShapeDtypeStruct((N_PAD, 1), jnp.float32),
            jax.ShapeDtypeStruct((N_PAD, 1), jnp.float32),
            jax.ShapeDtypeStruct((N_PAD, F), jnp.float32),
        ),
    )(deg2, features)


def _tc_mid(a1p, normsq):
    def body(a_ref, nsq_ref, a1_ref, g1_ref):
        a1 = a_ref[0] + a_ref[1]
        a1_ref[...] = a1
        g1_ref[...] = a1 * nsq_ref[...]

    return pl.pallas_call(
        body,
        grid=(N_PAD // RB,),
        in_specs=[
            pl.BlockSpec((NC, RB, F), lambda i: (0, i, 0)),
            pl.BlockSpec((RB, 1), lambda i: (i, 0)),
        ],
        out_specs=[
            pl.BlockSpec((RB, F), lambda i: (i, 0)),
            pl.BlockSpec((RB, F), lambda i: (i, 0)),
        ],
        out_shape=(
            jax.ShapeDtypeStruct((N_PAD, F), jnp.float32),
            jax.ShapeDtypeStruct((N_PAD, F), jnp.float32),
        ),
    )(a1p, normsq)


def _tc_layer(x, a1, a2p, norm, Wa, Wb, Wc, b, relu, want_g):
    fout = Wa.shape[1]

    def body(x_ref, a1_ref, a2_ref, n_ref, wa_ref, wb_ref, wc_ref, b_ref, *outs):
        nrm = n_ref[...]
        z1 = a1_ref[...] * nrm
        z2 = (a2_ref[0] + a2_ref[1]) * nrm
        acc = jnp.dot(x_ref[...], wa_ref[...], preferred_element_type=jnp.float32,
                      precision=lax.Precision.HIGHEST)
        acc += jnp.dot(z1, wb_ref[...], preferred_element_type=jnp.float32,
                       precision=lax.Precision.HIGHEST)
        acc += jnp.dot(z2, wc_ref[...], preferred_element_type=jnp.float32,
                       precision=lax.Precision.HIGHEST)
        acc += b_ref[...]
        if relu:
            acc = jnp.maximum(acc, 0.0)
        outs[0][...] = acc
        if want_g:
            outs[1][...] = acc * nrm

    out_shape = [jax.ShapeDtypeStruct((N_PAD, fout), jnp.float32)]
    if want_g:
        out_shape.append(jax.ShapeDtypeStruct((N_PAD, fout), jnp.float32))

    fin = x.shape[1]
    return pl.pallas_call(
        body,
        grid=(N_PAD // RB,),
        in_specs=[
            pl.BlockSpec((RB, fin), lambda i: (i, 0)),
            pl.BlockSpec((RB, F), lambda i: (i, 0)),
            pl.BlockSpec((NC, RB, F), lambda i: (0, i, 0)),
            pl.BlockSpec((RB, 1), lambda i: (i, 0)),
            pl.BlockSpec((fin, fout), lambda i: (0, 0)),
            pl.BlockSpec((F, fout), lambda i: (0, 0)),
            pl.BlockSpec((F, fout), lambda i: (0, 0)),
            pl.BlockSpec((1, fout), lambda i: (0, 0)),
        ],
        out_specs=[pl.BlockSpec((RB, fout), lambda i: (i, 0))
                   for _ in out_shape],
        out_shape=tuple(out_shape),
    )(x, a1, a2p, norm, Wa, Wb, Wc, b)


def kernel(features, edge_index, W0, b0, W1, b1, W2, b2):
    src = edge_index[0].astype(jnp.int32)
    dst = edge_index[1].astype(jnp.int32)
    npad_e = E_PAD - E
    pad_idx = (N + (jnp.arange(npad_e, dtype=jnp.int32) % PAD_ROWS))
    src_pad = jnp.concatenate([src, pad_idx])
    dst_pad = jnp.concatenate([dst, pad_idx]).reshape(NW, N_CHUNKS, CHUNK)

    ones_blk = jnp.ones((CHUNK, F), jnp.float32)
    zeros128 = jnp.zeros((ROWS_PER_SUB, F), jnp.float32)

    deg2 = _sc_degree(dst_pad, ones_blk, zeros128)
    norm, normsq, g = _tc_prep(deg2, features)

    x = jnp.pad(features, ((0, N_PAD - N), (0, 0)))
    params = [(W0, b0, True), (W1, b1, True), (W2, b2, False)]
    for li, (W, b, relu) in enumerate(params):
        fin = x.shape[1]
        Wa, Wb, Wc = W[0:fin], W[fin:2 * fin], W[2 * fin:3 * fin]
        a1p = _sc_propagate(g, src_pad, dst_pad, zeros128)
        a1, g1 = _tc_mid(a1p, normsq)
        a2p = _sc_propagate(g1, src_pad, dst_pad, zeros128)
        want_g = li < 2
        outs = _tc_layer(x, a1, a2p, norm, Wa, Wb, Wc, b.reshape(1, -1),
                         relu, want_g)
        if want_g:
            x, g = outs
        else:
            x = outs[0]
    return x[:N]
